# Initial kernel scaffold; baseline (speedup 1.0000x reference)
import jax
import jax.numpy as jnp
from jax.experimental import pallas as pl


def kernel(x0, edge_index, W1, b1, W2, b2):
    raise NotImplementedError("write your pallas kernel here")


# speedup vs baseline: 59.1486x; 59.1486x over previous
import functools

import jax
import jax.numpy as jnp
from jax import lax
from jax.experimental import pallas as pl
from jax.experimental.pallas import tpu as pltpu
from jax.experimental.pallas import tpu_sc as plsc

F = 8
P = 16
ROW_W = 128
K = 8
NC = 2
NS = 16
NW = NC * NS


def _sc_mesh():
    return plsc.VectorSubcoreMesh(core_axis_name="c", subcore_axis_name="s")


def _partition(wid, u_per, rem):
    n_u = jnp.where(wid < rem, u_per + 1, u_per)
    start_u = wid * u_per + jnp.minimum(wid, rem)
    return start_u, n_u


def _build_deg_kernel(u_per, rem, n_pad, n_t):
    @functools.partial(
        pl.kernel,
        out_type=jax.ShapeDtypeStruct((NC, n_pad, F), jnp.float32),
        mesh=_sc_mesh(),
        compiler_params=pltpu.CompilerParams(use_tc_tiling_on_sc=False),
        scratch_types=[
            pltpu.VMEM_SHARED((n_pad, F), jnp.float32),
            pltpu.VMEM((ROW_W, F), jnp.float32),
            pltpu.VMEM((K, ROW_W), jnp.int32),
            pltpu.SemaphoreType.DMA,
        ],
    )
    def deg_k(dst_hbm, zeros_hbm, ones_hbm, out_hbm,
              deg_sh, ones_v, idx_v, sem):
        c = lax.axis_index("c")
        s = lax.axis_index("s")
        wid = s * NC + c
        nslice = pl.ds(s * n_t, n_t)
        pltpu.sync_copy(zeros_hbm.at[nslice], deg_sh.at[nslice])
        pltpu.sync_copy(ones_hbm, ones_v)
        plsc.subcore_barrier()
        start_u, n_u = _partition(wid, u_per, rem)

        def step(g, carry):
            pltpu.sync_copy(dst_hbm.at[pl.ds((start_u + g) * K, K)], idx_v)
            descs = [
                pltpu.async_copy(ones_v, deg_sh.at[idx_v.at[j]], sem, add=True)
                for j in range(K)
            ]
            for d in descs:
                d.wait()
            return carry

        lax.fori_loop(0, n_u, step, 0)
        plsc.subcore_barrier()
        pltpu.sync_copy(deg_sh.at[nslice], out_hbm.at[c].at[nslice])

    return deg_k


def _build_edge_kernel(u_per, rem, n_pad, n_t):
    msg_rows = K * ROW_W

    @functools.partial(
        pl.kernel,
        out_type=jax.ShapeDtypeStruct((NC, n_pad, F), jnp.float32),
        mesh=_sc_mesh(),
        compiler_params=pltpu.CompilerParams(use_tc_tiling_on_sc=False),
        scratch_types=[
            pltpu.VMEM_SHARED((n_pad, F), jnp.float32),
            pltpu.VMEM((msg_rows, F), jnp.float32),
            pltpu.VMEM((K, ROW_W), jnp.int32),
            pltpu.VMEM((K, ROW_W), jnp.int32),
            pltpu.SemaphoreType.DMA,
            pltpu.SemaphoreType.DMA,
        ],
    )
    def edge_k(src_hbm, dst_hbm, g_hbm, out_hbm,
               acc_sh, msg_v, src_v, dst_v, gsem, ssem):
        c = lax.axis_index("c")
        s = lax.axis_index("s")
        wid = s * NC + c
        nslice = pl.ds(s * n_t, n_t)
        pltpu.sync_copy(g_hbm.at[nslice], acc_sh.at[nslice])
        plsc.subcore_barrier()
        start_u, n_u = _partition(wid, u_per, rem)

        def step(g, carry):
            base = (start_u + g) * K
            pltpu.sync_copy(src_hbm.at[pl.ds(base, K)], src_v)
            pltpu.sync_copy(dst_hbm.at[pl.ds(base, K)], dst_v)
            gd = [
                pltpu.async_copy(
                    g_hbm.at[src_v.at[j]],
                    msg_v.at[pl.ds(j * ROW_W, ROW_W)],
                    gsem,
                )
                for j in range(K)
            ]
            for d in gd:
                d.wait()
            sd = [
                pltpu.async_copy(
                    msg_v.at[pl.ds(j * ROW_W, ROW_W)],
                    acc_sh.at[dst_v.at[j]],
                    ssem,
                    add=True,
                )
                for j in range(K)
            ]
            for d in sd:
                d.wait()
            return carry

        lax.fori_loop(0, n_u, step, 0)
        plsc.subcore_barrier()
        pltpu.sync_copy(acc_sh.at[nslice], out_hbm.at[c].at[nslice])

    return edge_k


def kernel(x0, edge_index, W1, b1, W2, b2):
    N = x0.shape[0]
    E = edge_index.shape[1]
    f32 = jnp.float32

    n_pad = -(-N // (P * ROW_W)) * (P * ROW_W)
    if n_pad == N:
        n_pad += P * ROW_W
    n_t = n_pad // NS
    n_pk = n_pad // P

    rows = -(-E // ROW_W)
    units = -(-rows // K)
    rows_pad = units * K
    e_pad = rows_pad * ROW_W
    u_per = units // NW
    rem = units % NW

    if e_pad == E:
        src_rows = edge_index[0].reshape(rows_pad, ROW_W)
        dst_rows = edge_index[1].reshape(rows_pad, ROW_W)
    else:
        fill = jnp.full((e_pad - E,), N, dtype=jnp.int32)
        src_rows = jnp.concatenate([edge_index[0], fill]).reshape(rows_pad, ROW_W)
        dst_rows = jnp.concatenate([edge_index[1], fill]).reshape(rows_pad, ROW_W)

    x0pk = (
        jnp.zeros((n_pad, F), f32).at[:N, : x0.shape[1]].set(x0)
        .reshape(n_pk, P * F)
    )
    eye = jnp.eye(P, dtype=f32)
    W1bd = jnp.kron(eye, jnp.zeros((F, F), f32).at[: W1.shape[0], : W1.shape[1]].set(W1))
    W2bd = jnp.kron(eye, jnp.zeros((F, F), f32).at[: W2.shape[0], : W2.shape[1]].set(W2))
    b1pk = jnp.tile(jnp.zeros((F,), f32).at[: b1.shape[0]].set(b1), P)
    b2pk = jnp.tile(jnp.zeros((F,), f32).at[: b2.shape[0]].set(b2), P)
    b1t = jnp.tile(b1pk[None, :], (8, 1))
    b2t = jnp.tile(b2pk[None, :], (8, 1))
    zeros_nf = jnp.zeros((n_pad, F), f32)
    ones_rw = jnp.ones((ROW_W, F), f32)

    deg_k = _build_deg_kernel(u_per, rem, n_pad, n_t)
    deg2 = deg_k(dst_rows, zeros_nf, ones_rw)
    deg2pk = deg2.reshape(NC, n_pk, P * F)

    BP = 784
    grid = n_pk // BP
    assert grid * BP == n_pk

    def tc_a_body(degA_ref, degB_ref, x_ref, w_ref, dinv_ref, g_ref):
        d8 = lax.rsqrt(degA_ref[...] + degB_ref[...] + 1.0)
        dinv_ref[...] = d8
        h = jnp.dot(x_ref[...], w_ref[...], preferred_element_type=f32)
        g_ref[...] = h * d8

    blk = pl.BlockSpec((BP, P * F), lambda i: (i, 0))
    wblk = pl.BlockSpec((P * F, P * F), lambda i: (0, 0))
    bblk = pl.BlockSpec((8, P * F), lambda i: (0, 0))
    shp = jax.ShapeDtypeStruct((n_pk, P * F), f32)

    dinv_pk, g1pk = pl.pallas_call(
        tc_a_body,
        grid=(grid,),
        in_specs=[blk, blk, blk, wblk],
        out_specs=[blk, blk],
        out_shape=[shp, shp],
    )(deg2pk[0], deg2pk[1], x0pk, W1bd)

    edge_k = _build_edge_kernel(u_per, rem, n_pad, n_t)
    g1 = g1pk.reshape(n_pad, F)
    acc1 = edge_k(src_rows, dst_rows, g1)
    acc1pk = acc1.reshape(NC, n_pk, P * F)

    def tc_b_body(aA_ref, aB_ref, g_ref, dinv_ref, w_ref, b_ref, x_ref, g2_ref):
        tot = aA_ref[...] + aB_ref[...] - g_ref[...]
        db = dinv_ref[...]
        x1 = tot * db + b_ref[0:1, :]
        x_ref[...] = x1
        g2_ref[...] = jnp.dot(x1, w_ref[...], preferred_element_type=f32) * db

    x1pk, g2pk = pl.pallas_call(
        tc_b_body,
        grid=(grid,),
        in_specs=[blk, blk, blk, blk, wblk, bblk],
        out_specs=[blk, blk],
        out_shape=[shp, shp],
    )(acc1pk[0], acc1pk[1], g1pk, dinv_pk, W2bd, b1t)

    g2 = g2pk.reshape(n_pad, F)
    acc2 = edge_k(src_rows, dst_rows, g2)
    acc2pk = acc2.reshape(NC, n_pk, P * F)

    def tc_c_body(aA_ref, aB_ref, g_ref, dinv_ref, b_ref, x_ref):
        x_ref[...] = (aA_ref[...] + aB_ref[...] - g_ref[...]) * dinv_ref[...] \
            + b_ref[0:1, :]

    x2pk = pl.pallas_call(
        tc_c_body,
        grid=(grid,),
        in_specs=[blk, blk, blk, blk, bblk],
        out_specs=blk,
        out_shape=shp,
    )(acc2pk[0], acc2pk[1], g2pk, dinv_pk, b2t)

    x1 = x1pk.reshape(n_pad, F)[:N, : W1.shape[1]]
    x2 = x2pk.reshape(n_pad, F)[:N, : W2.shape[1]]
    return (x1, x2)


# speedup vs baseline: 136.3080x; 2.3045x over previous
import functools

import jax
import jax.numpy as jnp
from jax import lax
from jax.experimental import pallas as pl
from jax.experimental.pallas import tpu as pltpu
from jax.experimental.pallas import tpu_sc as plsc

F = 8
P = 16
ROW_W = 128
UR = 8
NC = 2
NS = 16
NW = NC * NS


def _sc_mesh():
    return plsc.VectorSubcoreMesh(core_axis_name="c", subcore_axis_name="s")


def _partition(wid, u_per, rem):
    start_u = wid * u_per + jnp.minimum(wid, rem)
    return start_u


def _build_deg_kernel(u_per, rem, n_pad, n_t):
    @functools.partial(
        pl.kernel,
        out_type=[
            jax.ShapeDtypeStruct((n_pad, F), jnp.float32),
            jax.ShapeDtypeStruct((n_pad, F), jnp.float32),
        ],
        mesh=_sc_mesh(),
        compiler_params=pltpu.CompilerParams(use_tc_tiling_on_sc=False),
        scratch_types=[
            pltpu.VMEM_SHARED((n_pad, F), jnp.float32),
            pltpu.VMEM((ROW_W, F), jnp.float32),
            pltpu.VMEM((2 * UR, ROW_W), jnp.int32),
            pltpu.SemaphoreType.DMA,
        ],
    )
    def deg_k(dst_hbm, zeros_hbm, ones_hbm, outA, outB,
              deg_sh, ones_v, idx_v, sem):
        c = lax.axis_index("c")
        s = lax.axis_index("s")
        wid = s * NC + c
        nslice = pl.ds(s * n_t, n_t)
        pltpu.sync_copy(zeros_hbm.at[nslice], deg_sh.at[nslice])
        pltpu.sync_copy(ones_hbm, ones_v)
        plsc.subcore_barrier()
        start_u = _partition(wid, u_per, rem)

        def scatter_rows(lo, hi):
            descs = [
                pltpu.async_copy(ones_v, deg_sh.at[idx_v.at[j]], sem, add=True)
                for j in range(lo, hi)
            ]
            for d in descs:
                d.wait()

        def step(g, carry):
            base = (start_u + 2 * g) * UR
            pltpu.sync_copy(dst_hbm.at[pl.ds(base, 2 * UR)], idx_v)
            scatter_rows(0, 2 * UR)
            return carry

        lax.fori_loop(0, u_per // 2, step, 0)

        def tail(u_idx):
            base = (start_u + u_idx) * UR
            pltpu.sync_copy(dst_hbm.at[pl.ds(base, UR)], idx_v.at[pl.ds(0, UR)])
            scatter_rows(0, UR)

        if u_per % 2 == 1:
            tail(u_per - 1)

        @pl.when(wid < rem)
        def _():
            tail(u_per)

        plsc.subcore_barrier()

        @pl.when(c == 0)
        def _():
            pltpu.sync_copy(deg_sh.at[nslice], outA.at[nslice])

        @pl.when(c == 1)
        def _():
            pltpu.sync_copy(deg_sh.at[nslice], outB.at[nslice])

    return deg_k


def _build_edge_kernel(u_per, rem, n_pad, n_t):
    msg_rows = 2 * UR * ROW_W

    @functools.partial(
        pl.kernel,
        out_type=[
            jax.ShapeDtypeStruct((n_pad, F), jnp.float32),
            jax.ShapeDtypeStruct((n_pad, F), jnp.float32),
        ],
        mesh=_sc_mesh(),
        compiler_params=pltpu.CompilerParams(use_tc_tiling_on_sc=False),
        scratch_types=[
            pltpu.VMEM_SHARED((n_pad, F), jnp.float32),
            pltpu.VMEM_SHARED((n_pad, F), jnp.float32),
            pltpu.VMEM((msg_rows, F), jnp.float32),
            pltpu.VMEM((2 * UR, ROW_W), jnp.int32),
            pltpu.VMEM((2 * UR, ROW_W), jnp.int32),
            pltpu.SemaphoreType.DMA,
            pltpu.SemaphoreType.DMA,
        ],
    )
    def edge_k(src_hbm, dst_hbm, g_hbm, outA, outB,
               g_sh, acc_sh, msg_v, src_v, dst_v, gsem, ssem):
        c = lax.axis_index("c")
        s = lax.axis_index("s")
        wid = s * NC + c
        nslice = pl.ds(s * n_t, n_t)
        pltpu.sync_copy(g_hbm.at[nslice], g_sh.at[nslice])
        pltpu.sync_copy(g_hbm.at[nslice], acc_sh.at[nslice])
        plsc.subcore_barrier()
        start_u = _partition(wid, u_per, rem)

        def move_rows(lo, hi):
            gd = [
                pltpu.async_copy(
                    g_sh.at[src_v.at[j]],
                    msg_v.at[pl.ds(j * ROW_W, ROW_W)],
                    gsem,
                )
                for j in range(lo, hi)
            ]
            for d in gd:
                d.wait()
            sd = [
                pltpu.async_copy(
                    msg_v.at[pl.ds(j * ROW_W, ROW_W)],
                    acc_sh.at[dst_v.at[j]],
                    ssem,
                    add=True,
                )
                for j in range(lo, hi)
            ]
            for d in sd:
                d.wait()

        def step(g, carry):
            base = (start_u + 2 * g) * UR
            pltpu.sync_copy(src_hbm.at[pl.ds(base, 2 * UR)], src_v)
            pltpu.sync_copy(dst_hbm.at[pl.ds(base, 2 * UR)], dst_v)
            move_rows(0, 2 * UR)
            return carry

        lax.fori_loop(0, u_per // 2, step, 0)

        def tail(u_idx):
            base = (start_u + u_idx) * UR
            pltpu.sync_copy(src_hbm.at[pl.ds(base, UR)], src_v.at[pl.ds(0, UR)])
            pltpu.sync_copy(dst_hbm.at[pl.ds(base, UR)], dst_v.at[pl.ds(0, UR)])
            move_rows(0, UR)

        if u_per % 2 == 1:
            tail(u_per - 1)

        @pl.when(wid < rem)
        def _():
            tail(u_per)

        plsc.subcore_barrier()

        @pl.when(c == 0)
        def _():
            pltpu.sync_copy(acc_sh.at[nslice], outA.at[nslice])

        @pl.when(c == 1)
        def _():
            pltpu.sync_copy(acc_sh.at[nslice], outB.at[nslice])

    return edge_k


def kernel(x0, edge_index, W1, b1, W2, b2):
    N = x0.shape[0]
    E = edge_index.shape[1]
    F1 = W1.shape[1]
    F2 = W2.shape[1]
    f32 = jnp.float32

    n_pad = -(-N // (P * ROW_W)) * (P * ROW_W)
    if n_pad == N:
        n_pad += P * ROW_W
    n_t = n_pad // NS
    n_pk = n_pad // P

    rows = -(-E // ROW_W)
    units = -(-rows // UR)
    rows_pad = units * UR
    e_pad = rows_pad * ROW_W
    u_per = units // NW
    rem = units % NW

    if e_pad == E:
        src_rows = edge_index[0].reshape(rows_pad, ROW_W)
        dst_rows = edge_index[1].reshape(rows_pad, ROW_W)
    else:
        fill = jnp.full((e_pad - E,), N, dtype=jnp.int32)
        src_rows = jnp.concatenate([edge_index[0], fill]).reshape(rows_pad, ROW_W)
        dst_rows = jnp.concatenate([edge_index[1], fill]).reshape(rows_pad, ROW_W)

    x0pk = (
        jnp.zeros((n_pad, F), f32).at[:N, : x0.shape[1]].set(x0)
        .reshape(n_pk, P * F)
    )
    eye = jnp.eye(P, dtype=f32)
    W1bd = jnp.kron(eye, jnp.zeros((F, F), f32).at[: W1.shape[0], :F1].set(W1))
    W2bd = jnp.kron(eye, jnp.zeros((F, F), f32).at[: W2.shape[0], :F2].set(W2))
    b1t = jnp.tile(jnp.zeros((F,), f32).at[:F1].set(b1), (8, P))
    b2t = jnp.tile(jnp.zeros((F,), f32).at[:F2].set(b2), (8, P))
    zeros_nf = jnp.zeros((n_pad, F), f32)
    ones_rw = jnp.ones((ROW_W, F), f32)

    deg_k = _build_deg_kernel(u_per, rem, n_pad, n_t)
    degA2d, degB2d = deg_k(dst_rows, zeros_nf, ones_rw)
    degA = degA2d.reshape(n_pk, P * F)
    degB = degB2d.reshape(n_pk, P * F)

    BP = n_pk // 8
    grid = 8

    def tc_a_body(degA_ref, degB_ref, x_ref, w_ref, dinv_ref, g_ref):
        d8 = lax.rsqrt(degA_ref[...] + degB_ref[...] + 1.0)
        dinv_ref[...] = d8
        h = jnp.dot(x_ref[...], w_ref[...], preferred_element_type=f32)
        g_ref[...] = h * d8

    blk = pl.BlockSpec((BP, P * F), lambda i: (i, 0))
    wblk = pl.BlockSpec((P * F, P * F), lambda i: (0, 0))
    bblk = pl.BlockSpec((8, P * F), lambda i: (0, 0))
    shp = jax.ShapeDtypeStruct((n_pk, P * F), f32)

    dinv_pk, g1pk = pl.pallas_call(
        tc_a_body,
        grid=(grid,),
        in_specs=[blk, blk, blk, wblk],
        out_specs=[blk, blk],
        out_shape=[shp, shp],
    )(degA, degB, x0pk, W1bd)

    edge_k = _build_edge_kernel(u_per, rem, n_pad, n_t)
    acc1A2d, acc1B2d = edge_k(src_rows, dst_rows, g1pk.reshape(n_pad, F))
    acc1A = acc1A2d.reshape(n_pk, P * F)
    acc1B = acc1B2d.reshape(n_pk, P * F)

    def tc_b_body(aA_ref, aB_ref, g_ref, dinv_ref, w_ref, b_ref,
                  x1_ref, g2_ref):
        tot = aA_ref[...] + aB_ref[...] - g_ref[...]
        db = dinv_ref[...]
        x1 = tot * db + b_ref[0:1, :]
        x1_ref[...] = x1
        g2_ref[...] = jnp.dot(x1, w_ref[...], preferred_element_type=f32) * db

    x1pk, g2pk = pl.pallas_call(
        tc_b_body,
        grid=(grid,),
        in_specs=[blk, blk, blk, blk, wblk, bblk],
        out_specs=[blk, blk],
        out_shape=[shp, shp],
    )(acc1A, acc1B, g1pk, dinv_pk, W2bd, b1t)

    acc2A2d, acc2B2d = edge_k(src_rows, dst_rows, g2pk.reshape(n_pad, F))
    acc2A = acc2A2d.reshape(n_pk, P * F)
    acc2B = acc2B2d.reshape(n_pk, P * F)

    def tc_c_body(aA_ref, aB_ref, g_ref, dinv_ref, b_ref, x2_ref):
        x2_ref[...] = (aA_ref[...] + aB_ref[...] - g_ref[...]) * dinv_ref[...] \
            + b_ref[0:1, :]

    x2pk = pl.pallas_call(
        tc_c_body,
        grid=(grid,),
        in_specs=[blk, blk, blk, blk, bblk],
        out_specs=blk,
        out_shape=shp,
    )(acc2A, acc2B, g2pk, dinv_pk, b2t)

    x1 = x1pk.reshape(n_pad, F)[:N, :F1]
    x2 = x2pk.reshape(n_pad, F)[:N, :F2]
    return (x1, x2)
